# baseline (speedup 1.0000x reference)
import jax
import jax.numpy as jnp
from jax.experimental import pallas as pl
from jax.experimental.pallas import tpu as pltpu

TBN = 16384


def _mlp_t_kernel(x_ref, w1t_ref, b1t_ref, w2t_ref, b2t_ref, w3t_ref,
                  b3t_ref, w4t_ref, b4_ref, o_ref):
    a = jnp.dot(w1t_ref[...], x_ref[...], preferred_element_type=jnp.float32)
    a = jnp.maximum(a + b1t_ref[...], 0.0)
    a = jnp.dot(w2t_ref[...], a, preferred_element_type=jnp.float32)
    a = jnp.maximum(a + b2t_ref[...], 0.0)
    a = jnp.dot(w3t_ref[...], a, preferred_element_type=jnp.float32)
    a = jnp.maximum(a + b3t_ref[...], 0.0)
    a = jnp.dot(w4t_ref[...], a, preferred_element_type=jnp.float32)
    o_ref[...] = jax.nn.sigmoid(a + b4_ref[...])


def kernel(x, w1, b1, w2, b2, w3, b3, w4, b4):
    B, f_in = x.shape
    n_out = w4.shape[1]

    xt = x.T
    w1t, b1t = w1.T, b1.T
    w2t, b2t = w2.T, b2.T
    w3t, b3t = w3.T, b3.T
    w4t = w4.T

    tbn = min(TBN, B)
    n_blocks = pl.cdiv(B, tbn)

    def const(arr):
        return pl.BlockSpec(arr.shape, lambda i: (0,) * arr.ndim)

    ot = pl.pallas_call(
        _mlp_t_kernel,
        out_shape=jax.ShapeDtypeStruct((n_out, B), jnp.float32),
        grid=(n_blocks,),
        in_specs=[pl.BlockSpec((f_in, tbn), lambda i: (0, i)),
                  const(w1t), const(b1t),
                  const(w2t), const(b2t),
                  const(w3t), const(b3t),
                  const(w4t), const(b4)],
        out_specs=pl.BlockSpec((n_out, tbn), lambda i: (0, i)),
        compiler_params=pltpu.CompilerParams(
            dimension_semantics=("parallel",),
            vmem_limit_bytes=48 * 1024 * 1024,
        ),
    )(xt, w1t, b1t, w2t, b2t, w3t, b3t, w4t, b4)

    return ot.reshape(B, n_out)


# speedup vs baseline: 1.2550x; 1.2550x over previous
import jax
import jax.numpy as jnp
from jax.experimental import pallas as pl
from jax.experimental.pallas import tpu as pltpu

TBN = 65536


def _mlp_t_kernel(x_ref, w1t_ref, b1t_ref, w2t_ref, b2t_ref, w3t_ref,
                  b3t_ref, w4t_ref, b4_ref, o_ref):
    a = jnp.dot(w1t_ref[...], x_ref[...], preferred_element_type=jnp.float32)
    a = jnp.maximum(a + b1t_ref[...], 0.0)
    a = jnp.dot(w2t_ref[...], a, preferred_element_type=jnp.float32)
    a = jnp.maximum(a + b2t_ref[...], 0.0)
    a = jnp.dot(w3t_ref[...], a, preferred_element_type=jnp.float32)
    a = jnp.maximum(a + b3t_ref[...], 0.0)
    a = jnp.dot(w4t_ref[...], a, preferred_element_type=jnp.float32)
    o_ref[...] = jax.nn.sigmoid(a + b4_ref[...])


def kernel(x, w1, b1, w2, b2, w3, b3, w4, b4):
    B, f_in = x.shape
    n_out = w4.shape[1]

    xt = x.T
    w1t, b1t = w1.T, b1.T
    w2t, b2t = w2.T, b2.T
    w3t, b3t = w3.T, b3.T
    w4t = w4.T

    tbn = min(TBN, B)
    n_blocks = pl.cdiv(B, tbn)

    def const(arr):
        return pl.BlockSpec(arr.shape, lambda i: (0,) * arr.ndim)

    ot = pl.pallas_call(
        _mlp_t_kernel,
        out_shape=jax.ShapeDtypeStruct((n_out, B), jnp.float32),
        grid=(n_blocks,),
        in_specs=[pl.BlockSpec((f_in, tbn), lambda i: (0, i)),
                  const(w1t), const(b1t),
                  const(w2t), const(b2t),
                  const(w3t), const(b3t),
                  const(w4t), const(b4)],
        out_specs=pl.BlockSpec((n_out, tbn), lambda i: (0, i)),
        compiler_params=pltpu.CompilerParams(
            dimension_semantics=("parallel",),
            vmem_limit_bytes=48 * 1024 * 1024,
        ),
    )(xt, w1t, b1t, w2t, b2t, w3t, b3t, w4t, b4)

    return ot.reshape(B, n_out)
